# baseline (speedup 1.0000x reference)
import functools
import jax
import jax.numpy as jnp
from jax import lax
from jax.experimental import pallas as pl
from jax.experimental.pallas import tpu as pltpu
from jax.experimental.pallas import tpu_sc as plsc

T, B, D = 4096, 16, 512
HALF = D // 2
NV = HALF // 16
CH = 64
NINF = float("-inf")

BT = 128
S0 = 2048
NBT = S0 // BT


def _sc_body(inp_hbm, len_hbm, out_hbm, len_v, pfx_s, buf0, buf1,
             accS, accM, shared, comb, outbuf, sem0, sem1):
    c = lax.axis_index("c")
    s = lax.axis_index("s")
    d0 = c * HALF

    pltpu.sync_copy(len_hbm, len_v.at[pl.ds(0, 16)])
    pfx_s[0] = 0
    tot = jnp.int32(0)
    for bb in range(B):
        lbb = len_v[pl.ds(bb, 16)][0]
        tot = tot + jnp.maximum(0, lbb - S0)
        pfx_s[bb + 1] = tot

    G = (tot + 15) // 16
    lo = s * G
    hi = jnp.minimum(tot, lo + G)

    zero = jnp.zeros((16,), jnp.float32)
    ninf = jnp.full((16,), NINF, jnp.float32)

    def init_body(bb, _):
        for g in range(NV):
            sl = pl.ds(g * 16, 16)
            accS[0, 0, bb, sl] = zero
            accM[0, 0, bb, sl] = ninf
        return 0

    lax.fori_loop(0, B, init_body, 0)

    bufs = (buf0, buf1)
    sems = (sem0, sem1)

    def batch_body(bb, _):
        p0 = pfx_s[bb]
        p1 = pfx_s[bb + 1]
        a = jnp.maximum(lo, p0)
        e = jnp.minimum(hi, p1)

        @pl.when(a < e)
        def _():
            seg = e - a
            tbase = S0 + (a - p0)
            nck = (seg + (CH - 1)) // CH
            npad = ((nck + 1) // 2) * 2

            def start_chunk(k, par):
                t0 = jnp.minimum(tbase + k * CH, T - CH)
                pltpu.make_async_copy(
                    inp_hbm.at[pl.ds(t0, CH), pl.ds(bb, 1), pl.ds(d0, HALF)],
                    bufs[par], sems[par],
                ).start()

            def wait_chunk(par):
                pltpu.make_async_copy(
                    inp_hbm.at[pl.ds(0, CH), pl.ds(0, 1), pl.ds(d0, HALF)],
                    bufs[par], sems[par],
                ).wait()

            start_chunk(0, 0)
            start_chunk(1, 1)

            init = tuple([zero] * NV + [ninf] * NV)

            def pair_body(p, carry):
                for par in range(2):
                    k = 2 * p + par
                    wait_chunk(par)
                    v = jnp.maximum(0, jnp.minimum(CH, seg - k * CH))
                    buf = bufs[par]

                    def row_body(r, carry2):
                        accs = list(carry2)
                        for g in range(NV):
                            x = buf[r, 0, pl.ds(g * 16, 16)]
                            accs[g] = accs[g] + x
                            accs[NV + g] = jnp.maximum(accs[NV + g], x)
                        return tuple(accs)

                    carry = plsc.parallel_loop(0, v, 1, unroll=4,
                                               carry=carry)(row_body)

                    @pl.when(k + 2 < npad)
                    def _():
                        start_chunk(k + 2, par)

                return carry

            accs = lax.fori_loop(0, npad // 2, pair_body, init)

            for g in range(NV):
                sl = pl.ds(g * 16, 16)
                accS[0, 0, bb, sl] = accs[g]
                accM[0, 0, bb, sl] = accs[NV + g]

        return 0

    lax.fori_loop(0, B, batch_body, 0)

    pltpu.sync_copy(accS, shared.at[pl.ds(s, 1), pl.ds(0, 1)])
    pltpu.sync_copy(accM, shared.at[pl.ds(s, 1), pl.ds(1, 1)])
    plsc.subcore_barrier()

    pltpu.sync_copy(shared.at[:, :, pl.ds(s, 1), :], comb)

    lb = len_v[pl.ds(s, 16)][0]

    pltpu.async_copy(
        inp_hbm.at[pl.ds(lb - 1, 1), pl.ds(s, 1), pl.ds(d0, HALF)],
        buf0.at[pl.ds(0, 1)], sem0,
    ).wait()

    for g in range(NV):
        sl = pl.ds(g * 16, 16)
        ssum = comb[0, 0, 0, sl]
        smax = comb[0, 1, 0, sl]
        for w in range(1, 16):
            ssum = ssum + comb[w, 0, 0, sl]
            smax = jnp.maximum(smax, comb[w, 1, 0, sl])
        outbuf[0, sl] = buf0[0, 0, sl]
        outbuf[1, sl] = ssum
        outbuf[2, sl] = smax

    for i in range(3):
        pltpu.sync_copy(
            outbuf.at[pl.ds(i, 1), :],
            out_hbm.at[pl.ds(s, 1), pl.ds(i * D + d0, HALF)],
        )


def _sc_tail(input, lengths):
    mesh = plsc.VectorSubcoreMesh(core_axis_name="c", subcore_axis_name="s")
    run = functools.partial(
        pl.kernel,
        mesh=mesh,
        out_type=jax.ShapeDtypeStruct((B, 3 * D), jnp.float32),
        scratch_types=[
            pltpu.VMEM((32,), jnp.int32),
            pltpu.SMEM((32,), jnp.int32),
            pltpu.VMEM((CH, 1, HALF), jnp.float32),
            pltpu.VMEM((CH, 1, HALF), jnp.float32),
            pltpu.VMEM((1, 1, B, HALF), jnp.float32),
            pltpu.VMEM((1, 1, B, HALF), jnp.float32),
            pltpu.VMEM_SHARED((16, 2, B, HALF), jnp.float32),
            pltpu.VMEM((16, 2, 1, HALF), jnp.float32),
            pltpu.VMEM((3, HALF), jnp.float32),
            pltpu.SemaphoreType.DMA,
            pltpu.SemaphoreType.DMA,
        ],
    )(_sc_body)
    return run(input, lengths)


def _tc_body(len_ref, x_ref, sum_ref, max_ref):
    i = pl.program_id(0)
    t0 = i * BT
    x = x_ref[...]
    trow = lax.broadcasted_iota(jnp.int32, (BT, B, 1), 0) + t0
    mask = trow < len_ref[...]
    psum = jnp.sum(jnp.where(mask, x, 0.0), axis=0)
    pmax = jnp.max(jnp.where(mask, x, NINF), axis=0)

    @pl.when(i == 0)
    def _():
        sum_ref[...] = psum
        max_ref[...] = pmax

    @pl.when(i > 0)
    def _():
        sum_ref[...] = sum_ref[...] + psum
        max_ref[...] = jnp.maximum(max_ref[...], pmax)


def _tc_prefix(input, lengths2d):
    return pl.pallas_call(
        _tc_body,
        grid=(NBT,),
        in_specs=[
            pl.BlockSpec((1, B, 1), lambda i: (0, 0, 0)),
            pl.BlockSpec((BT, B, D), lambda i: (i, 0, 0)),
        ],
        out_specs=[
            pl.BlockSpec((B, D), lambda i: (0, 0)),
            pl.BlockSpec((B, D), lambda i: (0, 0)),
        ],
        out_shape=[
            jax.ShapeDtypeStruct((B, D), jnp.float32),
            jax.ShapeDtypeStruct((B, D), jnp.float32),
        ],
    )(lengths2d, input)


def _comb_body(sc_ref, tsum_ref, tmax_ref, len_ref, out_ref):
    lenf = len_ref[...].astype(jnp.float32)
    out_ref[:, 0:D] = sc_ref[:, 0:D]
    out_ref[:, D:2 * D] = (sc_ref[:, D:2 * D] + tsum_ref[...]) / lenf
    out_ref[:, 2 * D:3 * D] = jnp.maximum(sc_ref[:, 2 * D:3 * D],
                                          tmax_ref[...])


def _combine(sc_out, tc_sum, tc_max, lengths_col):
    return pl.pallas_call(
        _comb_body,
        out_shape=jax.ShapeDtypeStruct((B, 3 * D), jnp.float32),
    )(sc_out, tc_sum, tc_max, lengths_col)


def kernel(input, lengths):
    sc_out = _sc_tail(input, lengths)
    tc_sum, tc_max = _tc_prefix(input, lengths.reshape(1, B, 1))
    return _combine(sc_out, tc_sum, tc_max, lengths.reshape(B, 1))


# speedup vs baseline: 1.0579x; 1.0579x over previous
import functools
import jax
import jax.numpy as jnp
from jax import lax
from jax.experimental import pallas as pl
from jax.experimental.pallas import tpu as pltpu
from jax.experimental.pallas import tpu_sc as plsc

T, B, D = 4096, 16, 512
HALF = D // 2
NV = HALF // 16
CH = 64
NINF = float("-inf")

BT = 256
S0 = 2048
NBT = S0 // BT


def _sc_body(inp_hbm, len_hbm, out_hbm, len_v, pfx_s, buf0, buf1,
             accS, accM, shared, comb, outbuf, sem0, sem1):
    c = lax.axis_index("c")
    s = lax.axis_index("s")
    d0 = c * HALF

    pltpu.sync_copy(len_hbm, len_v.at[pl.ds(0, 16)])
    pfx_s[0] = 0
    tot = jnp.int32(0)
    for bb in range(B):
        lbb = len_v[pl.ds(bb, 16)][0]
        tot = tot + jnp.maximum(0, lbb - S0)
        pfx_s[bb + 1] = tot

    G = (tot + 15) // 16
    lo = s * G
    hi = jnp.minimum(tot, lo + G)

    zero = jnp.zeros((16,), jnp.float32)
    ninf = jnp.full((16,), NINF, jnp.float32)

    def init_body(bb, _):
        for g in range(NV):
            sl = pl.ds(g * 16, 16)
            accS[0, 0, bb, sl] = zero
            accM[0, 0, bb, sl] = ninf
        return 0

    lax.fori_loop(0, B, init_body, 0)

    bufs = (buf0, buf1)
    sems = (sem0, sem1)

    def batch_body(bb, _):
        p0 = pfx_s[bb]
        p1 = pfx_s[bb + 1]
        a = jnp.maximum(lo, p0)
        e = jnp.minimum(hi, p1)

        @pl.when(a < e)
        def _():
            seg = e - a
            tbase = S0 + (a - p0)
            nck = (seg + (CH - 1)) // CH
            npad = ((nck + 1) // 2) * 2

            def start_chunk(k, par):
                t0 = jnp.minimum(tbase + k * CH, T - CH)
                pltpu.make_async_copy(
                    inp_hbm.at[pl.ds(t0, CH), pl.ds(bb, 1), pl.ds(d0, HALF)],
                    bufs[par], sems[par],
                ).start()

            def wait_chunk(par):
                pltpu.make_async_copy(
                    inp_hbm.at[pl.ds(0, CH), pl.ds(0, 1), pl.ds(d0, HALF)],
                    bufs[par], sems[par],
                ).wait()

            start_chunk(0, 0)
            start_chunk(1, 1)

            init = tuple([zero] * NV + [ninf] * NV)

            def pair_body(p, carry):
                for par in range(2):
                    k = 2 * p + par
                    wait_chunk(par)
                    v = jnp.maximum(0, jnp.minimum(CH, seg - k * CH))
                    buf = bufs[par]

                    def row_body(r, carry2):
                        accs = list(carry2)
                        for g in range(NV):
                            x = buf[r, 0, pl.ds(g * 16, 16)]
                            accs[g] = accs[g] + x
                            accs[NV + g] = jnp.maximum(accs[NV + g], x)
                        return tuple(accs)

                    carry = plsc.parallel_loop(0, v, 1, unroll=4,
                                               carry=carry)(row_body)

                    @pl.when(k + 2 < npad)
                    def _():
                        start_chunk(k + 2, par)

                return carry

            accs = lax.fori_loop(0, npad // 2, pair_body, init)

            for g in range(NV):
                sl = pl.ds(g * 16, 16)
                accS[0, 0, bb, sl] = accs[g]
                accM[0, 0, bb, sl] = accs[NV + g]

        return 0

    lax.fori_loop(0, B, batch_body, 0)

    pltpu.sync_copy(accS, shared.at[pl.ds(s, 1), pl.ds(0, 1)])
    pltpu.sync_copy(accM, shared.at[pl.ds(s, 1), pl.ds(1, 1)])
    plsc.subcore_barrier()

    pltpu.sync_copy(shared.at[:, :, pl.ds(s, 1), :], comb)

    lb = len_v[pl.ds(s, 16)][0]

    pltpu.async_copy(
        inp_hbm.at[pl.ds(lb - 1, 1), pl.ds(s, 1), pl.ds(d0, HALF)],
        buf0.at[pl.ds(0, 1)], sem0,
    ).wait()

    for g in range(NV):
        sl = pl.ds(g * 16, 16)
        ssum = comb[0, 0, 0, sl]
        smax = comb[0, 1, 0, sl]
        for w in range(1, 16):
            ssum = ssum + comb[w, 0, 0, sl]
            smax = jnp.maximum(smax, comb[w, 1, 0, sl])
        outbuf[0, sl] = buf0[0, 0, sl]
        outbuf[1, sl] = ssum
        outbuf[2, sl] = smax

    for i in range(3):
        pltpu.sync_copy(
            outbuf.at[pl.ds(i, 1), :],
            out_hbm.at[pl.ds(s, 1), pl.ds(i * D + d0, HALF)],
        )


def _sc_tail(input, lengths):
    mesh = plsc.VectorSubcoreMesh(core_axis_name="c", subcore_axis_name="s")
    run = functools.partial(
        pl.kernel,
        mesh=mesh,
        out_type=jax.ShapeDtypeStruct((B, 3 * D), jnp.float32),
        scratch_types=[
            pltpu.VMEM((32,), jnp.int32),
            pltpu.SMEM((32,), jnp.int32),
            pltpu.VMEM((CH, 1, HALF), jnp.float32),
            pltpu.VMEM((CH, 1, HALF), jnp.float32),
            pltpu.VMEM((1, 1, B, HALF), jnp.float32),
            pltpu.VMEM((1, 1, B, HALF), jnp.float32),
            pltpu.VMEM_SHARED((16, 2, B, HALF), jnp.float32),
            pltpu.VMEM((16, 2, 1, HALF), jnp.float32),
            pltpu.VMEM((3, HALF), jnp.float32),
            pltpu.SemaphoreType.DMA,
            pltpu.SemaphoreType.DMA,
        ],
    )(_sc_body)
    return run(input, lengths)


def _tc_body(len_ref, x_ref, sum_ref, max_ref):
    i = pl.program_id(0)
    t0 = i * BT
    x = x_ref[...]
    trow = lax.broadcasted_iota(jnp.int32, (BT, B, 1), 0) + t0
    mask = trow < len_ref[...]
    maskf = mask.astype(jnp.float32)
    psum = jnp.sum(x * maskf, axis=0)
    pmax = jnp.max(jnp.where(mask, x, NINF), axis=0)

    @pl.when(i == 0)
    def _():
        sum_ref[...] = psum
        max_ref[...] = pmax

    @pl.when(i > 0)
    def _():
        sum_ref[...] = sum_ref[...] + psum
        max_ref[...] = jnp.maximum(max_ref[...], pmax)


def _tc_prefix(input, lengths2d):
    return pl.pallas_call(
        _tc_body,
        grid=(NBT,),
        in_specs=[
            pl.BlockSpec((1, B, 1), lambda i: (0, 0, 0)),
            pl.BlockSpec((BT, B, D), lambda i: (i, 0, 0)),
        ],
        out_specs=[
            pl.BlockSpec((B, D), lambda i: (0, 0)),
            pl.BlockSpec((B, D), lambda i: (0, 0)),
        ],
        out_shape=[
            jax.ShapeDtypeStruct((B, D), jnp.float32),
            jax.ShapeDtypeStruct((B, D), jnp.float32),
        ],
    )(lengths2d, input)


def _comb_body(sc_ref, tsum_ref, tmax_ref, len_ref, out_ref):
    lenf = len_ref[...].astype(jnp.float32)
    out_ref[:, 0:D] = sc_ref[:, 0:D]
    out_ref[:, D:2 * D] = (sc_ref[:, D:2 * D] + tsum_ref[...]) / lenf
    out_ref[:, 2 * D:3 * D] = jnp.maximum(sc_ref[:, 2 * D:3 * D],
                                          tmax_ref[...])


def _combine(sc_out, tc_sum, tc_max, lengths_col):
    return pl.pallas_call(
        _comb_body,
        out_shape=jax.ShapeDtypeStruct((B, 3 * D), jnp.float32),
    )(sc_out, tc_sum, tc_max, lengths_col)


def kernel(input, lengths):
    sc_out = _sc_tail(input, lengths)
    tc_sum, tc_max = _tc_prefix(input, lengths.reshape(1, B, 1))
    return _combine(sc_out, tc_sum, tc_max, lengths.reshape(B, 1))
